# baseline (device time: 39063 ns/iter reference)
import jax
import jax.numpy as jnp
from jax import lax
from jax.experimental import pallas as pl
from jax.experimental.pallas import tpu as pltpu

B, SQ, H, D = 8, 1, 8, 64
SKV = 512
SCALE = D ** -0.5


def _body(q_ref, k_ref, v_ref, out_ref,
          loc_o, loc_m, loc_l, peer_o, peer_m, peer_l,
          send_sems, recv_sems):
    my_x = lax.axis_index("x")
    my_y = lax.axis_index("y")
    nbr = (my_x, 1 - my_y)

    barrier = pltpu.get_barrier_semaphore()
    pl.semaphore_signal(barrier, inc=1, device_id=nbr,
                        device_id_type=pl.DeviceIdType.MESH)
    pl.semaphore_wait(barrier, 1)

    q = q_ref[:, 0, :, :]
    k = k_ref[...]
    v = v_ref[...]

    s = jnp.sum(q[:, None, :, :] * k, axis=-1) * SCALE
    m = jnp.max(s, axis=1)
    p = jnp.exp(s - m[:, None, :])
    l = jnp.sum(p, axis=1)
    o = jnp.sum(p[:, :, :, None] * v, axis=1)

    loc_o[...] = o
    loc_m[...] = m
    loc_l[...] = l

    copies = [
        pltpu.make_async_remote_copy(
            src_ref=src, dst_ref=dst,
            send_sem=send_sems.at[i], recv_sem=recv_sems.at[i],
            device_id=nbr, device_id_type=pl.DeviceIdType.MESH,
        )
        for i, (src, dst) in enumerate(
            [(loc_o, peer_o), (loc_m, peer_m), (loc_l, peer_l)]
        )
    ]
    for c in copies:
        c.start()
    for c in copies:
        c.wait()

    m_new = jnp.maximum(loc_m[...], peer_m[...])
    a_loc = jnp.exp(loc_m[...] - m_new)
    a_peer = jnp.exp(peer_m[...] - m_new)
    l_new = a_loc * loc_l[...] + a_peer * peer_l[...]
    o_new = (a_loc[:, :, None] * loc_o[...] +
             a_peer[:, :, None] * peer_o[...]) / l_new[:, :, None]
    out_ref[...] = o_new[:, None, :, :]


def kernel(Q, K, V):
    return pl.pallas_call(
        _body,
        out_shape=jax.ShapeDtypeStruct((B, SQ, H, D), jnp.float32),
        in_specs=[pl.BlockSpec(memory_space=pltpu.VMEM)] * 3,
        out_specs=pl.BlockSpec(memory_space=pltpu.VMEM),
        scratch_shapes=[
            pltpu.VMEM((B, H, D), jnp.float32),
            pltpu.VMEM((B, H), jnp.float32),
            pltpu.VMEM((B, H), jnp.float32),
            pltpu.VMEM((B, H, D), jnp.float32),
            pltpu.VMEM((B, H), jnp.float32),
            pltpu.VMEM((B, H), jnp.float32),
            pltpu.SemaphoreType.DMA((3,)),
            pltpu.SemaphoreType.DMA((3,)),
        ],
        compiler_params=pltpu.CompilerParams(collective_id=0),
    )(Q, K, V)


# device time: 37937 ns/iter; 1.0297x vs baseline; 1.0297x over previous
import jax
import jax.numpy as jnp
from jax import lax
from jax.experimental import pallas as pl
from jax.experimental.pallas import tpu as pltpu

B, SQ, H, D = 8, 1, 8, 64
SKV = 512
HD = H * D
BK = B * SKV
SCALE = D ** -0.5
F32 = jnp.float32


def _body(q_ref, k_ref, v_ref, out_ref,
          loc_o, loc_m, loc_l, peer_o, peer_m, peer_l,
          send_sems, recv_sems):
    my_x = lax.axis_index("x")
    my_y = lax.axis_index("y")
    nbr = (my_x, 1 - my_y)

    barrier = pltpu.get_barrier_semaphore()
    pl.semaphore_signal(barrier, inc=1, device_id=nbr,
                        device_id_type=pl.DeviceIdType.MESH)
    pl.semaphore_wait(barrier, 1)

    hd_group = lax.broadcasted_iota(jnp.int32, (HD, H), 0) // D
    h_col = lax.broadcasted_iota(jnp.int32, (HD, H), 1)
    M = (hd_group == h_col).astype(F32)
    MT = M.T
    bk_group = lax.broadcasted_iota(jnp.int32, (B, BK), 1) // SKV
    b_row = lax.broadcasted_iota(jnp.int32, (B, BK), 0)
    Msel = (bk_group == b_row).astype(F32)

    q = q_ref[...]
    k3 = k_ref[...]
    prod = (k3 * q[:, None, :]).reshape(BK, HD)
    s2 = jnp.dot(prod, M, preferred_element_type=F32) * SCALE
    s3 = s2.reshape(B, SKV, H)
    m = jnp.max(s3, axis=1)
    p3 = jnp.exp(s3 - m[:, None, :])
    l = jnp.sum(p3, axis=1)
    p2 = p3.reshape(BK, H)
    pbig = jnp.dot(p2, MT, preferred_element_type=F32)
    ov = pbig * v_ref[...]
    o2 = jnp.dot(Msel, ov, preferred_element_type=F32)

    loc_o[...] = o2
    loc_m[...] = m
    loc_l[...] = l

    copies = [
        pltpu.make_async_remote_copy(
            src_ref=src, dst_ref=dst,
            send_sem=send_sems.at[i], recv_sem=recv_sems.at[i],
            device_id=nbr, device_id_type=pl.DeviceIdType.MESH,
        )
        for i, (src, dst) in enumerate(
            [(loc_o, peer_o), (loc_m, peer_m), (loc_l, peer_l)]
        )
    ]
    for c in copies:
        c.start()
    for c in copies:
        c.wait()

    m_new = jnp.maximum(loc_m[...], peer_m[...])
    a_loc = jnp.exp(loc_m[...] - m_new)
    a_peer = jnp.exp(peer_m[...] - m_new)
    l_new = a_loc * loc_l[...] + a_peer * peer_l[...]
    abig_loc = jnp.dot(a_loc, MT, preferred_element_type=F32)
    abig_peer = jnp.dot(a_peer, MT, preferred_element_type=F32)
    lbig = jnp.dot(l_new, MT, preferred_element_type=F32)
    out_ref[...] = (abig_loc * loc_o[...] + abig_peer * peer_o[...]) / lbig


def kernel(Q, K, V):
    out2 = pl.pallas_call(
        _body,
        out_shape=jax.ShapeDtypeStruct((B, HD), F32),
        in_specs=[pl.BlockSpec(memory_space=pltpu.VMEM)] * 3,
        out_specs=pl.BlockSpec(memory_space=pltpu.VMEM),
        scratch_shapes=[
            pltpu.VMEM((B, HD), F32),
            pltpu.VMEM((B, H), F32),
            pltpu.VMEM((B, H), F32),
            pltpu.VMEM((B, HD), F32),
            pltpu.VMEM((B, H), F32),
            pltpu.VMEM((B, H), F32),
            pltpu.SemaphoreType.DMA((3,)),
            pltpu.SemaphoreType.DMA((3,)),
        ],
        compiler_params=pltpu.CompilerParams(collective_id=0),
    )(
        Q.reshape(B, HD),
        K.reshape(B, SKV, HD),
        V.reshape(BK, HD),
    )
    return out2.reshape(B, SQ, H, D)
